# baseline (speedup 1.0000x reference)
import functools

import jax
import jax.numpy as jnp
from jax import lax
from jax.experimental import pallas as pl
from jax.experimental.pallas import tpu as pltpu
from jax.experimental.pallas import tpu_sc as plsc

DIM = 128
K = 100000
BATCH = 16384

NC = 2
NCW = 16
RPW = DIM // NCW
CW = 1664
NCHUNK = 50
LASTW = K - BATCH - NCHUNK * CW

TW = 1024
NB = TW // DIM
PITCH = DIM + 8


def _sc_body(k_hbm, q_hbm, o_hbm, buf, lastbuf, kin, obuf,
             isem, osem, lsem, ksem, tsem):
    wid = lax.axis_index("s") * NC + lax.axis_index("c")

    @pl.when(wid < NCW)
    def _copy():
        r0 = pl.multiple_of(wid * RPW, RPW)

        def din(j):
            co = BATCH + j * CW
            return pltpu.make_async_copy(
                q_hbm.at[pl.ds(r0, RPW), pl.ds(co, CW)], buf.at[j % 2],
                isem.at[j % 2])

        def dout(j):
            co = BATCH + j * CW
            return pltpu.make_async_copy(
                buf.at[j % 2], o_hbm.at[pl.ds(r0, RPW), pl.ds(co, CW)],
                osem.at[j % 2])

        def lin():
            return pltpu.make_async_copy(
                q_hbm.at[pl.ds(r0, RPW), pl.ds(K - LASTW, LASTW)],
                lastbuf, lsem.at[0])

        def lout():
            return pltpu.make_async_copy(
                lastbuf, o_hbm.at[pl.ds(r0, RPW), pl.ds(K - LASTW, LASTW)],
                lsem.at[1])

        lin().start()
        din(0).start()
        for j in range(NCHUNK):
            if j + 1 < NCHUNK:
                if j >= 1:
                    dout(j - 1).wait()
                din(j + 1).start()
            din(j).wait()
            dout(j).start()
            if j == 1:
                lin().wait()
                lout().start()
        dout(NCHUNK - 2).wait()
        dout(NCHUNK - 1).wait()
        lout().wait()

    @pl.when(wid >= NCW)
    def _xpose():
        c0 = pl.multiple_of((wid - NCW) * TW, TW)
        iota = lax.iota(jnp.int32, 16)

        def kin_dma(b):
            return pltpu.make_async_copy(
                k_hbm.at[pl.ds(c0 + b * DIM, DIM), :],
                kin.at[b % 2, :, pl.ds(0, DIM)], ksem.at[b % 2])

        def obuf_dma(b):
            return pltpu.make_async_copy(
                obuf.at[b % 2], o_hbm.at[:, pl.ds(c0 + b * DIM, DIM)],
                tsem.at[b % 2])

        row_idx = [iota + (16 * g) for g in range(8)]

        kin_dma(0).start()
        for b in range(NB):
            if b + 1 < NB:
                kin_dma(b + 1).start()
            kin_dma(b).wait()
            if b >= 2:
                obuf_dma(b - 2).wait()
            ksl = kin.at[b % 2]
            osl = obuf.at[b % 2]

            @plsc.parallel_loop(0, DIM, 1, unroll=8)
            def body(d):
                cols = jnp.full((16,), d, dtype=jnp.int32)
                for g in range(8):
                    v = plsc.load_gather(ksl, [row_idx[g], cols])
                    osl[d, pl.ds(16 * g, 16)] = v
            obuf_dma(b).start()
        obuf_dma(NB - 2).wait()
        obuf_dma(NB - 1).wait()


_sc_kernel = functools.partial(
    pl.kernel,
    out_type=jax.ShapeDtypeStruct((DIM, K), jnp.float32),
    mesh=plsc.VectorSubcoreMesh(core_axis_name="c", subcore_axis_name="s"),
    compiler_params=pltpu.CompilerParams(needs_layout_passes=False),
    scratch_types=[
        pltpu.VMEM((2, RPW, CW), jnp.float32),
        pltpu.VMEM((RPW, LASTW), jnp.float32),
        pltpu.VMEM((2, DIM, PITCH), jnp.float32),
        pltpu.VMEM((2, DIM, DIM), jnp.float32),
        pltpu.SemaphoreType.DMA((2,)),
        pltpu.SemaphoreType.DMA((2,)),
        pltpu.SemaphoreType.DMA((2,)),
        pltpu.SemaphoreType.DMA((2,)),
        pltpu.SemaphoreType.DMA((2,)),
    ],
)(_sc_body)


def kernel(keys, queue):
    new_queue = _sc_kernel(keys, queue)
    new_ptr = jnp.array([BATCH % K], dtype=jnp.int32)
    return new_queue, new_ptr


# speedup vs baseline: 1.0476x; 1.0476x over previous
import functools

import jax
import jax.numpy as jnp
from jax import lax
from jax.experimental import pallas as pl
from jax.experimental.pallas import tpu as pltpu
from jax.experimental.pallas import tpu_sc as plsc

DIM = 128
K = 100000
BATCH = 16384

NC = 2
NCW = 16
RPW = DIM // NCW
CW = 5248
NCHUNK = 15
LASTW = K - BATCH - NCHUNK * CW

TBLK = 2048
NTBLK = BATCH // TBLK


def _sc_copy_body(q_hbm, o_hbm, buf, lastbuf, isem, osem, lsem):
    wid = lax.axis_index("s") * NC + lax.axis_index("c")

    @pl.when(wid < NCW)
    def _():
        r0 = pl.multiple_of(wid * RPW, RPW)

        def din(j):
            co = BATCH + j * CW
            return pltpu.make_async_copy(
                q_hbm.at[pl.ds(r0, RPW), pl.ds(co, CW)], buf.at[j % 2],
                isem.at[j % 2])

        def dout(j):
            co = BATCH + j * CW
            return pltpu.make_async_copy(
                buf.at[j % 2], o_hbm.at[pl.ds(r0, RPW), pl.ds(co, CW)],
                osem.at[j % 2])

        def lin():
            return pltpu.make_async_copy(
                q_hbm.at[pl.ds(r0, RPW), pl.ds(K - LASTW, LASTW)],
                lastbuf, lsem.at[0])

        def lout():
            return pltpu.make_async_copy(
                lastbuf, o_hbm.at[pl.ds(r0, RPW), pl.ds(K - LASTW, LASTW)],
                lsem.at[1])

        lin().start()
        din(0).start()
        for j in range(NCHUNK):
            if j + 1 < NCHUNK:
                if j >= 1:
                    dout(j - 1).wait()
                din(j + 1).start()
            din(j).wait()
            dout(j).start()
            if j == 1:
                lin().wait()
                lout().start()
        dout(NCHUNK - 2).wait()
        dout(NCHUNK - 1).wait()
        lout().wait()


_sc_copy = functools.partial(
    pl.kernel,
    out_type=jax.ShapeDtypeStruct((DIM, K), jnp.float32),
    mesh=plsc.VectorSubcoreMesh(core_axis_name="c", subcore_axis_name="s"),
    scratch_types=[
        pltpu.VMEM((2, RPW, CW), jnp.float32),
        pltpu.VMEM((RPW, LASTW), jnp.float32),
        pltpu.SemaphoreType.DMA((2,)),
        pltpu.SemaphoreType.DMA((2,)),
        pltpu.SemaphoreType.DMA((2,)),
    ],
)(_sc_copy_body)


def _xpose_body(k_ref, _, o_ref):
    o_ref[...] = k_ref[...].T


def kernel(keys, queue):
    tail = _sc_copy(queue)

    new_queue = pl.pallas_call(
        _xpose_body,
        grid=(NTBLK,),
        in_specs=[
            pl.BlockSpec((TBLK, DIM), lambda i: (i, 0)),
            pl.BlockSpec(memory_space=pl.ANY),
        ],
        out_specs=pl.BlockSpec((DIM, TBLK), lambda i: (0, i)),
        out_shape=jax.ShapeDtypeStruct((DIM, K), jnp.float32),
        input_output_aliases={1: 0},
    )(keys, tail)

    new_ptr = jnp.array([BATCH % K], dtype=jnp.int32)
    return new_queue, new_ptr


# speedup vs baseline: 1.1200x; 1.0692x over previous
import functools

import jax
import jax.numpy as jnp
from jax import lax
from jax.experimental import pallas as pl
from jax.experimental.pallas import tpu as pltpu
from jax.experimental.pallas import tpu_sc as plsc

DIM = 128
K = 100000
BATCH = 16384

NC = 2
RPW = 8
CW = 3840
NFULL = 21
LASTW = K - BATCH - NFULL * CW
NL = 10

TBLK = 2048
NTBLK = BATCH // TBLK


def _sc_copy_body(q_hbm, o_hbm, buf, lastbuf, isem, osem, lsem):
    ch = lax.axis_index("c")
    r0 = pl.multiple_of(lax.axis_index("s") * RPW, RPW)

    def _co(l):
        return pl.multiple_of(BATCH + (2 * l + ch) * CW, 128)

    def din(l):
        return pltpu.make_async_copy(
            q_hbm.at[pl.ds(r0, RPW), pl.ds(_co(l), CW)], buf.at[l % 2],
            isem.at[l % 2])

    def dout(l):
        return pltpu.make_async_copy(
            buf.at[l % 2], o_hbm.at[pl.ds(r0, RPW), pl.ds(_co(l), CW)],
            osem.at[l % 2])

    def lin():
        return pltpu.make_async_copy(
            q_hbm.at[pl.ds(r0, RPW), pl.ds(K - LASTW, LASTW)],
            lastbuf, lsem.at[0])

    def lout():
        return pltpu.make_async_copy(
            lastbuf, o_hbm.at[pl.ds(r0, RPW), pl.ds(K - LASTW, LASTW)],
            lsem.at[1])

    din(0).start()
    for l in range(NL):
        if l + 1 < NL:
            if l >= 1:
                dout(l - 1).wait()
            din(l + 1).start()
        din(l).wait()
        dout(l).start()
    dout(NL - 2).wait()

    @pl.when(ch == 0)
    def _():
        din(NL).start()
        din(NL).wait()
        dout(NL).start()
        dout(NL).wait()

    @pl.when(ch == 1)
    def _():
        lin().start()
        lin().wait()
        lout().start()
        lout().wait()

    dout(NL - 1).wait()


_sc_copy = functools.partial(
    pl.kernel,
    out_type=jax.ShapeDtypeStruct((DIM, K), jnp.float32),
    mesh=plsc.VectorSubcoreMesh(core_axis_name="c", subcore_axis_name="s"),
    scratch_types=[
        pltpu.VMEM((2, RPW, CW), jnp.float32),
        pltpu.VMEM((RPW, LASTW), jnp.float32),
        pltpu.SemaphoreType.DMA((2,)),
        pltpu.SemaphoreType.DMA((2,)),
        pltpu.SemaphoreType.DMA((2,)),
    ],
)(_sc_copy_body)


def _xpose_body(k_ref, _, o_ref):
    o_ref[...] = k_ref[...].T


def kernel(keys, queue):
    tail = _sc_copy(queue)

    new_queue = pl.pallas_call(
        _xpose_body,
        grid=(NTBLK,),
        in_specs=[
            pl.BlockSpec((TBLK, DIM), lambda i: (i, 0)),
            pl.BlockSpec(memory_space=pl.ANY),
        ],
        out_specs=pl.BlockSpec((DIM, TBLK), lambda i: (0, i)),
        out_shape=jax.ShapeDtypeStruct((DIM, K), jnp.float32),
        input_output_aliases={1: 0},
    )(keys, tail)

    new_ptr = jnp.array([BATCH % K], dtype=jnp.int32)
    return new_queue, new_ptr
